# baseline (speedup 1.0000x reference)
import functools

import jax
import jax.numpy as jnp
from jax import lax
from jax.experimental import pallas as pl
from jax.experimental.pallas import tpu as pltpu
from jax.experimental.pallas import tpu_sc as plsc

B, S, D, E = 2, 2048, 1024, 8
T = B * S
EP = 128
M = 1024
MM = 512
NB = T // M
NBM = T // MM
WMAX = NBM + E - 1
XCOL = D + 128

NW = 32
TPW = T // NW
CS = 32
NCH = TPW // CS


def _gate_body(x_ref, wg_ref, tri_ref, tinc_ref, gs_ref, gs1_ref, eye_ref,
               gate_ref, rank_ref, xsc_ref, wl_ref, loss_ref,
               run_ref, psum_ref):
    b = pl.program_id(0)

    @pl.when(b == 0)
    def _():
        run_ref[...] = jnp.zeros_like(run_ref)
        psum_ref[...] = jnp.zeros_like(psum_ref)

    xb = x_ref[...]
    wg = wg_ref[...]
    logits = jnp.dot(xb, wg, preferred_element_type=jnp.float32)
    lane = lax.broadcasted_iota(jnp.int32, (M, EP), 1)
    valid = lane < E
    neg = jnp.full_like(logits, -jnp.inf)
    logit_m = jnp.where(valid, logits, neg)
    mx = jnp.max(logit_m, axis=-1, keepdims=True)
    ex = jnp.where(valid, jnp.exp(logit_m - mx), 0.0)
    den = jnp.sum(ex, axis=-1, keepdims=True)
    probs = ex / den
    gate = jnp.argmax(logit_m, axis=-1).astype(jnp.int32)
    selp = jnp.max(probs, axis=-1)

    onehot = jnp.where(lane == gate[:, None], 1.0, 0.0)
    cum_excl = jnp.dot(tri_ref[...], onehot, preferred_element_type=jnp.float32)
    local_rank = jnp.sum(cum_excl * onehot, axis=1)
    carry = jnp.sum(run_ref[...] * onehot, axis=1)
    rank = (local_rank + carry).astype(jnp.int32)

    gate_ref[...] = gate
    rank_ref[...] = rank
    run_ref[...] = run_ref[...] + jnp.sum(onehot, axis=0, keepdims=True)
    psum_ref[...] = psum_ref[...] + jnp.sum(probs, axis=0, keepdims=True)

    xsc_ref[:, :D] = xb * selp[:, None]
    lane2 = lax.broadcasted_iota(jnp.int32, (M, XCOL - D), 1)
    xsc_ref[:, D:] = jnp.where(lane2 == 0, selp[:, None], 0.0)

    @pl.when(b == NB - 1)
    def _():
        def exact_mm(vec, mat):
            hi = jnp.floor(vec * (1.0 / 128.0))
            lo = vec - hi * 128.0
            return (jnp.dot(hi, mat, preferred_element_type=jnp.float32)
                    * 128.0
                    + jnp.dot(lo, mat, preferred_element_type=jnp.float32))

        cnt = run_ref[...]
        psum = psum_ref[...]
        cum = exact_mm(cnt, tinc_ref[...])
        exc = cum - cnt
        pv = lax.broadcasted_iota(jnp.int32, (1, EP), 1)
        bbf = (pv >> 3).astype(jnp.float32)
        eef = (pv & 7).astype(jnp.float32)
        offe = exact_mm(exc, gs_ref[...])
        offe1 = exact_mm(exc, gs1_ref[...])
        seg_s = jnp.maximum(offe, bbf * MM)
        seg_e = jnp.minimum(offe1, bbf * MM + MM)
        act = jnp.logical_and(seg_e > seg_s, pv < NBM * E)
        actf = jnp.where(act, 1.0, 0.0)
        slot = jnp.dot(actf, tinc_ref[...],
                       preferred_element_type=jnp.float32) - 1.0
        slot_m = jnp.where(act, slot, -1.0)
        slot_t = jnp.sum(eye_ref[...] * slot_m, axis=1, keepdims=True)
        wlane = lax.broadcasted_iota(jnp.int32, (EP, EP), 1).astype(jnp.float32)
        match = jnp.where(slot_t == wlane, 1.0, 0.0)
        vals = jnp.concatenate([bbf, eef, seg_s, seg_e], axis=0)
        out4 = exact_mm(vals, match)
        cw = jnp.sum(match, axis=0, keepdims=True)
        filled = cw > 0.0
        blk_l = jnp.where(filled, out4[0:1, :], float(NBM - 1))
        eid_l = jnp.where(filled, out4[1:2, :], 0.0)
        rs_l = jnp.where(filled, out4[2:3, :], 0.0)
        re_l = jnp.where(filled, out4[3:4, :], 0.0)
        wl_ref[0:1, :] = blk_l.astype(jnp.int32)
        wl_ref[1:2, :] = eid_l.astype(jnp.int32)
        wl_ref[2:3, :] = rs_l.astype(jnp.int32)
        wl_ref[3:4, :] = re_l.astype(jnp.int32)
        wl_ref[4:5, :] = exc.astype(jnp.int32)
        wl_ref[5:6, :] = cnt.astype(jnp.int32)
        wl_ref[6:8, :] = jnp.zeros((2, EP), jnp.int32)
        ftot = cnt / jnp.float32(T)
        loss = jnp.float32(E) * jnp.sum((psum / jnp.float32(T)) * ftot)
        loss_ref[...] = jnp.full((1, EP), loss, jnp.float32)


def _pos_chunks(gate_hbm, rank_hbm, off_hbm, g_v, r_v, o_v, pos_v, base):
    pltpu.sync_copy(gate_hbm.at[pl.ds(base, TPW)], g_v)
    pltpu.sync_copy(rank_hbm.at[pl.ds(base, TPW)], r_v)
    pltpu.sync_copy(off_hbm, o_v)
    for c in range(TPW // 16):
        g16 = g_v[pl.ds(c * 16, 16)]
        off16 = plsc.load_gather(o_v, [g16])
        j, k = divmod(c * 16, CS)
        pos_v[j, pl.ds(k, 16)] = off16 + r_v[pl.ds(c * 16, 16)]


def _dispatch_body(gate_hbm, rank_hbm, off_hbm, xsc_hbm, xs_hbm,
                   g_v, r_v, o_v, pos_v, rows_v, rs0, rs1, ss0, ss1):
    wid = lax.axis_index("s") * 2 + lax.axis_index("c")
    base = wid * TPW
    rsem = (rs0, rs1)
    ssem = (ss0, ss1)

    def read(j):
        return pltpu.async_copy(xsc_hbm.at[pl.ds(base + j * CS, CS)],
                                rows_v.at[j % 2], rsem[j % 2])

    def scat(j):
        return pltpu.async_copy(rows_v.at[j % 2], xs_hbm.at[pos_v.at[j]],
                                ssem[j % 2])

    r0, r1 = read(0), read(1)
    _pos_chunks(gate_hbm, rank_hbm, off_hbm, g_v, r_v, o_v, pos_v, base)
    r0.wait(); s0 = scat(0)
    r1.wait(); s1 = scat(1)
    s0.wait(); r2 = read(2)
    r2.wait(); s2 = scat(2)
    s1.wait(); r3 = read(3)
    r3.wait(); s3 = scat(3)
    s2.wait(); s3.wait()


def _combine_body(gate_hbm, rank_hbm, off_hbm, ys_hbm, out_hbm,
                  g_v, r_v, o_v, pos_v, rows_v, rs0, rs1, ss0, ss1):
    wid = lax.axis_index("s") * 2 + lax.axis_index("c")
    base = wid * TPW
    gsem = (rs0, rs1)
    wsem = (ss0, ss1)

    def gath(j):
        return pltpu.async_copy(ys_hbm.at[pos_v.at[j]], rows_v.at[j % 2],
                                gsem[j % 2])

    def write(j):
        return pltpu.async_copy(rows_v.at[j % 2],
                                out_hbm.at[pl.ds(base + j * CS, CS)],
                                wsem[j % 2])

    _pos_chunks(gate_hbm, rank_hbm, off_hbm, g_v, r_v, o_v, pos_v, base)
    g0, g1 = gath(0), gath(1)
    g0.wait(); w0 = write(0)
    g1.wait(); w1 = write(1)
    w0.wait(); g2 = gath(2)
    g2.wait(); w2 = write(2)
    w1.wait(); g3 = gath(3)
    g3.wait(); w3 = write(3)
    w2.wait(); w3.wait()


@functools.cache
def _sc_kernels():
    mesh = plsc.VectorSubcoreMesh(core_axis_name="c", subcore_axis_name="s")
    params = pltpu.CompilerParams(needs_layout_passes=False)
    dispatch = pl.kernel(
        _dispatch_body, mesh=mesh, compiler_params=params,
        out_type=jax.ShapeDtypeStruct((T, XCOL), jnp.float32),
        scratch_types=[
            pltpu.VMEM((TPW,), jnp.int32),
            pltpu.VMEM((TPW,), jnp.int32),
            pltpu.VMEM((16,), jnp.int32),
            pltpu.VMEM((NCH, CS), jnp.int32),
            pltpu.VMEM((2, CS, XCOL), jnp.float32),
            pltpu.SemaphoreType.DMA,
            pltpu.SemaphoreType.DMA,
            pltpu.SemaphoreType.DMA,
            pltpu.SemaphoreType.DMA,
        ],
    )
    combine = pl.kernel(
        _combine_body, mesh=mesh, compiler_params=params,
        out_type=jax.ShapeDtypeStruct((T, D), jnp.float32),
        scratch_types=[
            pltpu.VMEM((TPW,), jnp.int32),
            pltpu.VMEM((TPW,), jnp.int32),
            pltpu.VMEM((16,), jnp.int32),
            pltpu.VMEM((NCH, CS), jnp.int32),
            pltpu.VMEM((2, CS, D), jnp.float32),
            pltpu.SemaphoreType.DMA,
            pltpu.SemaphoreType.DMA,
            pltpu.SemaphoreType.DMA,
            pltpu.SemaphoreType.DMA,
        ],
    )
    return dispatch, combine


def _moe_body(wl_s, xs_ref, w_ref, b_ref, ys_ref):
    w = pl.program_id(0)
    blk = wl_s[0, w]
    prev_blk = wl_s[0, jnp.maximum(w - 1, 0)]
    first = jnp.logical_or(w == 0, blk != prev_blk)
    xb = xs_ref[...]
    y = jnp.dot(xb[:, :D], w_ref[0], preferred_element_type=jnp.float32)
    y = y + xb[:, D:D + 1] * b_ref[0, 0, :][None, :]
    jg = blk * MM + lax.broadcasted_iota(jnp.int32, (MM, 1), 0)
    mask = jnp.logical_and(jg >= wl_s[2, w], jg < wl_s[3, w])
    contrib = jnp.where(mask, y, 0.0)
    ys_ref[...] = jnp.where(first, contrib, ys_ref[...] + contrib)


def kernel(x, attention_mask, W_gate, W_experts, b_experts):
    del attention_mask
    xf = x.reshape(T, D)
    wg_pad = jnp.zeros((D, EP), jnp.float32).at[:, :E].set(W_gate)
    tri = jnp.tril(jnp.ones((M, M), jnp.float32), -1)
    tinc = jnp.triu(jnp.ones((EP, EP), jnp.float32))
    jj = jnp.arange(EP, dtype=jnp.int32)[:, None]
    ppl = jnp.arange(EP, dtype=jnp.int32)[None, :]
    gs = ((jj == (ppl & 7)) & (ppl < NBM * E)).astype(jnp.float32)
    gs1 = ((jj == (ppl & 7) + 1) & (ppl < NBM * E)).astype(jnp.float32)
    eye = jnp.eye(EP, dtype=jnp.float32)

    gate, rank, xsc, wl, lossrow = pl.pallas_call(
        _gate_body,
        grid=(NB,),
        in_specs=[
            pl.BlockSpec((M, D), lambda b: (b, 0)),
            pl.BlockSpec((D, EP), lambda b: (0, 0)),
            pl.BlockSpec((M, M), lambda b: (0, 0)),
            pl.BlockSpec((EP, EP), lambda b: (0, 0)),
            pl.BlockSpec((EP, EP), lambda b: (0, 0)),
            pl.BlockSpec((EP, EP), lambda b: (0, 0)),
            pl.BlockSpec((EP, EP), lambda b: (0, 0)),
        ],
        out_specs=[
            pl.BlockSpec((M,), lambda b: (b,)),
            pl.BlockSpec((M,), lambda b: (b,)),
            pl.BlockSpec((M, XCOL), lambda b: (b, 0)),
            pl.BlockSpec((8, EP), lambda b: (0, 0)),
            pl.BlockSpec((1, EP), lambda b: (0, 0)),
        ],
        out_shape=[
            jax.ShapeDtypeStruct((T,), jnp.int32),
            jax.ShapeDtypeStruct((T,), jnp.int32),
            jax.ShapeDtypeStruct((T, XCOL), jnp.float32),
            jax.ShapeDtypeStruct((8, EP), jnp.int32),
            jax.ShapeDtypeStruct((1, EP), jnp.float32),
        ],
        scratch_shapes=[pltpu.VMEM((1, EP), jnp.float32),
                        pltpu.VMEM((1, EP), jnp.float32)],
    )(xf, wg_pad, tri, tinc, gs, gs1, eye)

    _dispatch, _combine = _sc_kernels()
    off_pad = wl[4, :16]
    xs = _dispatch(gate, rank, off_pad, xsc)

    grid_spec = pltpu.PrefetchScalarGridSpec(
        num_scalar_prefetch=1,
        grid=(WMAX,),
        in_specs=[
            pl.BlockSpec((MM, XCOL), lambda w, s: (s[0, w], 0)),
            pl.BlockSpec((1, D, D), lambda w, s: (s[1, w], 0, 0)),
            pl.BlockSpec((1, 1, D), lambda w, s: (s[1, w], 0, 0)),
        ],
        out_specs=pl.BlockSpec((MM, D), lambda w, s: (s[0, w], 0)),
    )
    ys = pl.pallas_call(
        _moe_body,
        grid_spec=grid_spec,
        out_shape=jax.ShapeDtypeStruct((T, D), jnp.float32),
    )(wl, xs, W_experts, b_experts.reshape(E, 1, D))

    out = _combine(gate, rank, off_pad, ys)

    balance_loss = lossrow[0, 0]
    gate_load = wl[5, :E]
    return out.reshape(B, S, D), balance_loss, gate_load


# speedup vs baseline: 1.0362x; 1.0362x over previous
import functools

import jax
import jax.numpy as jnp
from jax import lax
from jax.experimental import pallas as pl
from jax.experimental.pallas import tpu as pltpu
from jax.experimental.pallas import tpu_sc as plsc

B, S, D, E = 2, 2048, 1024, 8
T = B * S
EP = 128
M = 512
MM = 512
NB = T // M
NBM = T // MM
WMAX = NBM + E - 1
SCOL = 128

NW = 32
TPW = T // NW
CS = 32
NCH = TPW // CS


def _gate_body(x_ref, wg_ref, tri_ref, tinc_ref, gs_ref, gs1_ref, eye_ref,
               gate_ref, rank_ref, selp_ref, wl_ref, loss_ref,
               run_ref, psum_ref):
    b = pl.program_id(0)

    @pl.when(b == 0)
    def _():
        run_ref[...] = jnp.zeros_like(run_ref)
        psum_ref[...] = jnp.zeros_like(psum_ref)

    xb = x_ref[...]
    wg = wg_ref[...]
    logits = jnp.dot(xb, wg, preferred_element_type=jnp.float32)
    lane = lax.broadcasted_iota(jnp.int32, (M, EP), 1)
    valid = lane < E
    neg = jnp.full_like(logits, -jnp.inf)
    logit_m = jnp.where(valid, logits, neg)
    mx = jnp.max(logit_m, axis=-1, keepdims=True)
    ex = jnp.where(valid, jnp.exp(logit_m - mx), 0.0)
    den = jnp.sum(ex, axis=-1, keepdims=True)
    probs = ex / den
    gate = jnp.argmax(logit_m, axis=-1).astype(jnp.int32)
    selp = jnp.max(probs, axis=-1)

    onehot = jnp.where(lane == gate[:, None], 1.0, 0.0)
    cum_excl = jnp.dot(tri_ref[...], onehot, preferred_element_type=jnp.float32)
    local_rank = jnp.sum(cum_excl * onehot, axis=1)
    carry = jnp.sum(run_ref[...] * onehot, axis=1)
    rank = (local_rank + carry).astype(jnp.int32)

    gate_ref[...] = gate
    rank_ref[...] = rank
    run_ref[...] = run_ref[...] + jnp.sum(onehot, axis=0, keepdims=True)
    psum_ref[...] = psum_ref[...] + jnp.sum(probs, axis=0, keepdims=True)

    lane2 = lax.broadcasted_iota(jnp.int32, (M, SCOL), 1)
    selp_ref[...] = jnp.where(lane2 == 0, selp[:, None], 0.0)

    @pl.when(b == NB - 1)
    def _():
        def exact_mm(vec, mat):
            hi = jnp.floor(vec * (1.0 / 128.0))
            lo = vec - hi * 128.0
            return (jnp.dot(hi, mat, preferred_element_type=jnp.float32)
                    * 128.0
                    + jnp.dot(lo, mat, preferred_element_type=jnp.float32))

        cnt = run_ref[...]
        psum = psum_ref[...]
        cum = exact_mm(cnt, tinc_ref[...])
        exc = cum - cnt
        pv = lax.broadcasted_iota(jnp.int32, (1, EP), 1)
        bbf = (pv >> 3).astype(jnp.float32)
        eef = (pv & 7).astype(jnp.float32)
        offe = exact_mm(exc, gs_ref[...])
        offe1 = exact_mm(exc, gs1_ref[...])
        seg_s = jnp.maximum(offe, bbf * MM)
        seg_e = jnp.minimum(offe1, bbf * MM + MM)
        act = jnp.logical_and(seg_e > seg_s, pv < NBM * E)
        actf = jnp.where(act, 1.0, 0.0)
        slot = jnp.dot(actf, tinc_ref[...],
                       preferred_element_type=jnp.float32) - 1.0
        slot_m = jnp.where(act, slot, -1.0)
        slot_t = jnp.sum(eye_ref[...] * slot_m, axis=1, keepdims=True)
        wlane = lax.broadcasted_iota(jnp.int32, (EP, EP), 1).astype(jnp.float32)
        match = jnp.where(slot_t == wlane, 1.0, 0.0)
        vals = jnp.concatenate([bbf, eef, seg_s, seg_e], axis=0)
        out4 = exact_mm(vals, match)
        cw = jnp.sum(match, axis=0, keepdims=True)
        filled = cw > 0.0
        blk_l = jnp.where(filled, out4[0:1, :], float(NBM - 1))
        eid_l = jnp.where(filled, out4[1:2, :], 0.0)
        rs_l = jnp.where(filled, out4[2:3, :], 0.0)
        re_l = jnp.where(filled, out4[3:4, :], 0.0)
        wl_ref[0:1, :] = blk_l.astype(jnp.int32)
        wl_ref[1:2, :] = eid_l.astype(jnp.int32)
        wl_ref[2:3, :] = rs_l.astype(jnp.int32)
        wl_ref[3:4, :] = re_l.astype(jnp.int32)
        wl_ref[4:5, :] = exc.astype(jnp.int32)
        wl_ref[5:6, :] = cnt.astype(jnp.int32)
        wl_ref[6:8, :] = jnp.zeros((2, EP), jnp.int32)
        ftot = cnt / jnp.float32(T)
        loss = jnp.float32(E) * jnp.sum((psum / jnp.float32(T)) * ftot)
        loss_ref[...] = jnp.full((1, EP), loss, jnp.float32)


def _pos_chunks(gate_hbm, rank_hbm, off_hbm, g_v, r_v, o_v, pos_v, base):
    pltpu.sync_copy(gate_hbm.at[pl.ds(base, TPW)], g_v)
    pltpu.sync_copy(rank_hbm.at[pl.ds(base, TPW)], r_v)
    pltpu.sync_copy(off_hbm, o_v)
    for c in range(TPW // 16):
        g16 = g_v[pl.ds(c * 16, 16)]
        off16 = plsc.load_gather(o_v, [g16])
        j, k = divmod(c * 16, CS)
        pos_v[j, pl.ds(k, 16)] = off16 + r_v[pl.ds(c * 16, 16)]


def _dispatch_body(gate_hbm, rank_hbm, off_hbm, x_hbm, sp_hbm, xs_hbm, sps_hbm,
                   g_v, r_v, o_v, pos_v, rows_v, srow_v,
                   rs0, rs1, ss0, ss1, ps0, ps1, qs0, qs1):
    wid = lax.axis_index("s") * 2 + lax.axis_index("c")
    base = wid * TPW
    rsem = (rs0, rs1)
    ssem = (ss0, ss1)
    psem = (ps0, ps1)
    qsem = (qs0, qs1)

    def read(j):
        return (pltpu.async_copy(x_hbm.at[pl.ds(base + j * CS, CS)],
                                 rows_v.at[j % 2], rsem[j % 2]),
                pltpu.async_copy(sp_hbm.at[pl.ds(base + j * CS, CS)],
                                 srow_v.at[j % 2], psem[j % 2]))

    def scat(j):
        return (pltpu.async_copy(rows_v.at[j % 2], xs_hbm.at[pos_v.at[j]],
                                 ssem[j % 2]),
                pltpu.async_copy(srow_v.at[j % 2], sps_hbm.at[pos_v.at[j]],
                                 qsem[j % 2]))

    def waitall(hs):
        for h in hs:
            h.wait()

    r0, r1 = read(0), read(1)
    _pos_chunks(gate_hbm, rank_hbm, off_hbm, g_v, r_v, o_v, pos_v, base)
    waitall(r0); s0 = scat(0)
    waitall(r1); s1 = scat(1)
    waitall(s0); r2 = read(2)
    waitall(r2); s2 = scat(2)
    waitall(s1); r3 = read(3)
    waitall(r3); s3 = scat(3)
    waitall(s2); waitall(s3)


def _combine_body(gate_hbm, rank_hbm, off_hbm, ys_hbm, out_hbm,
                  g_v, r_v, o_v, pos_v, rows_v, rs0, rs1, ss0, ss1):
    wid = lax.axis_index("s") * 2 + lax.axis_index("c")
    base = wid * TPW
    gsem = (rs0, rs1)
    wsem = (ss0, ss1)

    def gath(j):
        return pltpu.async_copy(ys_hbm.at[pos_v.at[j]], rows_v.at[j % 2],
                                gsem[j % 2])

    def write(j):
        return pltpu.async_copy(rows_v.at[j % 2],
                                out_hbm.at[pl.ds(base + j * CS, CS)],
                                wsem[j % 2])

    _pos_chunks(gate_hbm, rank_hbm, off_hbm, g_v, r_v, o_v, pos_v, base)
    g0, g1 = gath(0), gath(1)
    g0.wait(); w0 = write(0)
    g1.wait(); w1 = write(1)
    w0.wait(); g2 = gath(2)
    g2.wait(); w2 = write(2)
    w1.wait(); g3 = gath(3)
    g3.wait(); w3 = write(3)
    w2.wait(); w3.wait()


@functools.cache
def _sc_kernels():
    mesh = plsc.VectorSubcoreMesh(core_axis_name="c", subcore_axis_name="s")
    params = pltpu.CompilerParams(needs_layout_passes=False)
    dispatch = pl.kernel(
        _dispatch_body, mesh=mesh, compiler_params=params,
        out_type=(jax.ShapeDtypeStruct((T, D), jnp.float32),
                  jax.ShapeDtypeStruct((T, SCOL), jnp.float32)),
        scratch_types=[
            pltpu.VMEM((TPW,), jnp.int32),
            pltpu.VMEM((TPW,), jnp.int32),
            pltpu.VMEM((16,), jnp.int32),
            pltpu.VMEM((NCH, CS), jnp.int32),
            pltpu.VMEM((2, CS, D), jnp.float32),
            pltpu.VMEM((2, CS, SCOL), jnp.float32),
            pltpu.SemaphoreType.DMA,
            pltpu.SemaphoreType.DMA,
            pltpu.SemaphoreType.DMA,
            pltpu.SemaphoreType.DMA,
            pltpu.SemaphoreType.DMA,
            pltpu.SemaphoreType.DMA,
            pltpu.SemaphoreType.DMA,
            pltpu.SemaphoreType.DMA,
        ],
    )
    combine = pl.kernel(
        _combine_body, mesh=mesh, compiler_params=params,
        out_type=jax.ShapeDtypeStruct((T, D), jnp.float32),
        scratch_types=[
            pltpu.VMEM((TPW,), jnp.int32),
            pltpu.VMEM((TPW,), jnp.int32),
            pltpu.VMEM((16,), jnp.int32),
            pltpu.VMEM((NCH, CS), jnp.int32),
            pltpu.VMEM((2, CS, D), jnp.float32),
            pltpu.SemaphoreType.DMA,
            pltpu.SemaphoreType.DMA,
            pltpu.SemaphoreType.DMA,
            pltpu.SemaphoreType.DMA,
        ],
    )
    return dispatch, combine


def _moe_body(wl_s, xs_ref, sp_ref, w_ref, b_ref, ys_ref):
    w = pl.program_id(0)
    blk = wl_s[0, w]
    prev_blk = wl_s[0, jnp.maximum(w - 1, 0)]
    first = jnp.logical_or(w == 0, blk != prev_blk)
    xb = xs_ref[...]
    y = jnp.dot(xb, w_ref[0], preferred_element_type=jnp.float32)
    y = (y + b_ref[0, 0, :][None, :]) * sp_ref[:, 0:1]
    jg = blk * MM + lax.broadcasted_iota(jnp.int32, (MM, 1), 0)
    mask = jnp.logical_and(jg >= wl_s[2, w], jg < wl_s[3, w])
    contrib = jnp.where(mask, y, 0.0)
    ys_ref[...] = jnp.where(first, contrib, ys_ref[...] + contrib)


def kernel(x, attention_mask, W_gate, W_experts, b_experts):
    del attention_mask
    xf = x.reshape(T, D)
    wg_pad = jnp.zeros((D, EP), jnp.float32).at[:, :E].set(W_gate)
    tri = jnp.tril(jnp.ones((M, M), jnp.float32), -1)
    tinc = jnp.triu(jnp.ones((EP, EP), jnp.float32))
    jj = jnp.arange(EP, dtype=jnp.int32)[:, None]
    ppl = jnp.arange(EP, dtype=jnp.int32)[None, :]
    gs = ((jj == (ppl & 7)) & (ppl < NBM * E)).astype(jnp.float32)
    gs1 = ((jj == (ppl & 7) + 1) & (ppl < NBM * E)).astype(jnp.float32)
    eye = jnp.eye(EP, dtype=jnp.float32)

    gate, rank, selp16, wl, lossrow = pl.pallas_call(
        _gate_body,
        grid=(NB,),
        in_specs=[
            pl.BlockSpec((M, D), lambda b: (b, 0)),
            pl.BlockSpec((D, EP), lambda b: (0, 0)),
            pl.BlockSpec((M, M), lambda b: (0, 0)),
            pl.BlockSpec((EP, EP), lambda b: (0, 0)),
            pl.BlockSpec((EP, EP), lambda b: (0, 0)),
            pl.BlockSpec((EP, EP), lambda b: (0, 0)),
            pl.BlockSpec((EP, EP), lambda b: (0, 0)),
        ],
        out_specs=[
            pl.BlockSpec((M,), lambda b: (b,)),
            pl.BlockSpec((M,), lambda b: (b,)),
            pl.BlockSpec((M, SCOL), lambda b: (b, 0)),
            pl.BlockSpec((8, EP), lambda b: (0, 0)),
            pl.BlockSpec((1, EP), lambda b: (0, 0)),
        ],
        out_shape=[
            jax.ShapeDtypeStruct((T,), jnp.int32),
            jax.ShapeDtypeStruct((T,), jnp.int32),
            jax.ShapeDtypeStruct((T, SCOL), jnp.float32),
            jax.ShapeDtypeStruct((8, EP), jnp.int32),
            jax.ShapeDtypeStruct((1, EP), jnp.float32),
        ],
        scratch_shapes=[pltpu.VMEM((1, EP), jnp.float32),
                        pltpu.VMEM((1, EP), jnp.float32)],
    )(xf, wg_pad, tri, tinc, gs, gs1, eye)

    _dispatch, _combine = _sc_kernels()
    off_pad = wl[4, :16]
    xs, sps = _dispatch(gate, rank, off_pad, xf, selp16)

    grid_spec = pltpu.PrefetchScalarGridSpec(
        num_scalar_prefetch=1,
        grid=(WMAX,),
        in_specs=[
            pl.BlockSpec((MM, D), lambda w, s: (s[0, w], 0)),
            pl.BlockSpec((MM, SCOL), lambda w, s: (s[0, w], 0)),
            pl.BlockSpec((1, D, D), lambda w, s: (s[1, w], 0, 0)),
            pl.BlockSpec((1, 1, D), lambda w, s: (s[1, w], 0, 0)),
        ],
        out_specs=pl.BlockSpec((MM, D), lambda w, s: (s[0, w], 0)),
    )
    ys = pl.pallas_call(
        _moe_body,
        grid_spec=grid_spec,
        out_shape=jax.ShapeDtypeStruct((T, D), jnp.float32),
    )(wl, xs, sps, W_experts, b_experts.reshape(E, 1, D))

    out = _combine(gate, rank, off_pad, ys)

    balance_loss = lossrow[0, 0]
    gate_load = wl[5, :E]
    return out.reshape(B, S, D), balance_loss, gate_load
